# baseline (speedup 1.0000x reference)
import functools

import jax
import jax.numpy as jnp
from jax import lax
from jax.experimental import pallas as pl
from jax.experimental.pallas import tpu as pltpu
from jax.experimental.pallas import tpu_sc as plsc

B = 16384
NF = 32
LW = 128
BW = 4096

NP_ROWS = 1000000
NQ_ROWS = 100000
NBLK_P = 31
NBLK_Q = 4
QS_P = NBLK_P * BW
QS_Q = NBLK_Q * BW

NC = 2
NS = 16
NW = NC * NS
BPW = B // NW
CH = 128
NCH = BPW // CH

_mesh = plsc.VectorSubcoreMesh(core_axis_name="c", subcore_axis_name="s")


def _bf16_round(iw):
    return iw + jnp.int32(0x8000)


def _precompute_body(x0, x1, x2, x3, x4, x5, x6, x7, w_ref, out_ref):
    lo4 = jnp.concatenate(
        [x0[...], x1[...], x2[...], x3[...]], axis=0).astype(jnp.bfloat16)
    hi4 = jnp.concatenate(
        [x4[...], x5[...], x6[...], x7[...]], axis=0).astype(jnp.bfloat16)
    e_lo = lax.dot_general(
        lo4, w_ref[...], dimension_numbers=(((0,), (0,)), ((), ())),
        preferred_element_type=jnp.float32)
    e_hi = lax.dot_general(
        hi4, w_ref[...], dimension_numbers=(((0,), (0,)), ((), ())),
        preferred_element_type=jnp.float32)
    ilo = _bf16_round(lax.bitcast_convert_type(e_lo, jnp.int32))
    ihi = _bf16_round(lax.bitcast_convert_type(e_hi, jnp.int32))
    word = jnp.bitwise_or(jnp.bitwise_and(ilo, jnp.int32(-65536)),
                          lax.shift_right_logical(ihi, 16))
    out_ref[...] = lax.bitcast_convert_type(word, jnp.float32)


def _make_precompute(nblk, n_cols):
    last = (n_cols + BW - 1) // BW - 1
    in_specs = [
        pl.BlockSpec((NF, BW), functools.partial(
            lambda u, i: (0, jnp.minimum(nblk * u + i, last)), u))
        for u in range(8)
    ]
    in_specs.append(pl.BlockSpec((LW, LW), lambda i: (0, 0)))
    return pl.pallas_call(
        _precompute_body,
        grid=(nblk,),
        in_specs=in_specs,
        out_specs=pl.BlockSpec((BW, LW), lambda i: (i, 0)),
        out_shape=jax.ShapeDtypeStruct((nblk * BW, LW), jnp.float32),
    )


_precompute_p = _make_precompute(NBLK_P, NP_ROWS)
_precompute_q = _make_precompute(NBLK_Q, NQ_ROWS)


def _to_packed(v, qs):
    u = jnp.where(v >= qs, 1, 0)
    for k in range(2, 8):
        u = u + jnp.where(v >= k * qs, 1, 0)
    return v - u * qs


def _make_gather(qs):
    @functools.partial(
        pl.kernel,
        mesh=_mesh,
        out_type=jax.ShapeDtypeStruct((B, LW), jnp.float32),
        scratch_types=[
            pltpu.VMEM((BPW,), jnp.int32),
            pltpu.VMEM((CH, LW), jnp.float32),
            pltpu.VMEM((CH, LW), jnp.float32),
            pltpu.VMEM((CH, LW), jnp.float32),
            pltpu.VMEM((CH, LW), jnp.float32),
            pltpu.SemaphoreType.DMA,
            pltpu.SemaphoreType.DMA,
        ],
    )
    def _g(idx_hbm, t_hbm, row_out, idx_v, b0, b1, b2, b3, gsem, wsem):
        wid = lax.axis_index("s") * NC + lax.axis_index("c")
        base = wid * BPW
        pltpu.sync_copy(idx_hbm.at[0, pl.ds(base, BPW)], idx_v)
        for s in range(BPW // 16):
            sl = pl.ds(s * 16, 16)
            idx_v[sl] = _to_packed(idx_v[sl], qs)
        bufs = (b0, b1, b2, b3)
        gcps = [
            pltpu.async_copy(t_hbm.at[idx_v.at[pl.ds(c * CH, CH)]],
                             bufs[c], gsem)
            for c in range(NCH)
        ]
        for cp in gcps:
            cp.wait()
        wcps = [
            pltpu.async_copy(bufs[c], row_out.at[pl.ds(base + c * CH, CH)],
                             wsem)
            for c in range(NCH)
        ]
        for cp in wcps:
            cp.wait()

    return _g


_gather_p = _make_gather(QS_P)
_gather_q = _make_gather(QS_Q)


NB = 4
BM = B // NB


def _chunk8(v, qs):
    u = jnp.where(v >= qs, 1, 0)
    for k in range(2, 8):
        u = u + jnp.where(v >= k * qs, 1, 0)
    return u


def _unpack_select(idx_row, row_ref, qs):
    chunks = lax.broadcasted_iota(jnp.int32, (8, 1), 0)
    oh = (chunks == _chunk8(idx_row, qs)).astype(jnp.float32)
    lane_blk = lax.broadcasted_iota(jnp.int32, (8, LW), 1) // NF
    is_top = (chunks < 4)
    r_top = ((lane_blk == chunks) & is_top).astype(jnp.float32)
    r_low = ((lane_blk == chunks - 4) & ~is_top).astype(jnp.float32)
    m_top = lax.dot_general(oh, r_top,
                            dimension_numbers=(((0,), (0,)), ((), ())),
                            preferred_element_type=jnp.float32)
    m_low = lax.dot_general(oh, r_low,
                            dimension_numbers=(((0,), (0,)), ((), ())),
                            preferred_element_type=jnp.float32)
    iw = lax.bitcast_convert_type(row_ref[...], jnp.int32)
    top = lax.bitcast_convert_type(
        jnp.bitwise_and(iw, jnp.int32(-65536)), jnp.float32)
    low = lax.bitcast_convert_type(lax.shift_left(iw, 16), jnp.float32)
    return top * m_top + low * m_low


def _mlp_body(u_ref, v_ref, p_ref, q_ref, s_ref, b1_ref, w2t_ref,
              b2_ref, out_ref):
    xp = _unpack_select(u_ref[...], p_ref, QS_P)
    xq = _unpack_select(v_ref[...], q_ref, QS_Q)
    h = lax.dot_general(xp + xq, s_ref[...],
                        dimension_numbers=(((1,), (0,)), ((), ())),
                        preferred_element_type=jnp.float32)
    h = jnp.maximum(h + b1_ref[...], 0.0)
    o = lax.dot_general(w2t_ref[...], h,
                        dimension_numbers=(((1,), (1,)), ((), ())),
                        preferred_element_type=jnp.float32)
    out_ref[...] = o + b2_ref[...]


_mlp = pl.pallas_call(
    _mlp_body,
    grid=(NB,),
    in_specs=[
        pl.BlockSpec((1, BM), lambda i: (0, i)),
        pl.BlockSpec((1, BM), lambda i: (0, i)),
        pl.BlockSpec((BM, LW), lambda i: (i, 0)),
        pl.BlockSpec((BM, LW), lambda i: (i, 0)),
        pl.BlockSpec((LW, NF), lambda i: (0, 0)),
        pl.BlockSpec((1, NF), lambda i: (0, 0)),
        pl.BlockSpec((1, NF), lambda i: (0, 0)),
        pl.BlockSpec((1, 1), lambda i: (0, 0)),
    ],
    out_specs=pl.BlockSpec((1, BM), lambda i: (0, i)),
    out_shape=jax.ShapeDtypeStruct((1, B), jnp.float32),
)


def kernel(user, product, P_table, Q_table, W1, b1, W2, b2):
    pt = P_table.T
    qt = Q_table.T
    eye4 = jnp.eye(4, dtype=jnp.float32)
    w4a = jnp.kron(eye4, W1[:NF]).astype(jnp.bfloat16)
    w4b = jnp.kron(eye4, W1[NF:]).astype(jnp.bfloat16)
    uidx = user.reshape(1, B).astype(jnp.int32)
    pidx = product.reshape(1, B).astype(jnp.int32)
    b_sc = _precompute_q(qt, qt, qt, qt, qt, qt, qt, qt, w4b)
    qrow = _gather_q(pidx, b_sc)
    pt_gated, _ = lax.optimization_barrier((pt, b_sc))
    a_sc = _precompute_p(pt_gated, pt_gated, pt_gated, pt_gated, pt_gated,
                         pt_gated, pt_gated, pt_gated, w4a)
    prow = _gather_p(uidx, a_sc)
    sel = jnp.tile(jnp.eye(NF, dtype=jnp.float32), (4, 1))
    out_t = _mlp(uidx, pidx, prow, qrow, sel, b1.reshape(1, NF),
                 W2.reshape(1, NF), b2.reshape(1, 1))
    return out_t.T


# speedup vs baseline: 1.0409x; 1.0409x over previous
import functools

import jax
import jax.numpy as jnp
from jax import lax
from jax.experimental import pallas as pl
from jax.experimental.pallas import tpu as pltpu
from jax.experimental.pallas import tpu_sc as plsc

B = 16384
NF = 32
LW = 128
BW = 4096

NP_ROWS = 1000000
NQ_ROWS = 100000
NBLK_P = 31
NBLK_Q = 4
QS_P = NBLK_P * BW
QS_Q = NBLK_Q * BW

NC = 2
NS = 16
NW = NC * NS
BPW = B // NW
CH = 128
NCH = BPW // CH

_mesh = plsc.VectorSubcoreMesh(core_axis_name="c", subcore_axis_name="s")


def _bf16_round(iw):
    return iw + jnp.int32(0x8000)


def _precompute_body(x0, x1, x2, x3, x4, x5, x6, x7, w_ref, out_ref):
    lo4 = jnp.concatenate(
        [x0[...], x1[...], x2[...], x3[...]], axis=0).astype(jnp.bfloat16)
    hi4 = jnp.concatenate(
        [x4[...], x5[...], x6[...], x7[...]], axis=0).astype(jnp.bfloat16)
    e_lo = lax.dot_general(
        lo4, w_ref[...], dimension_numbers=(((0,), (0,)), ((), ())),
        preferred_element_type=jnp.float32)
    e_hi = lax.dot_general(
        hi4, w_ref[...], dimension_numbers=(((0,), (0,)), ((), ())),
        preferred_element_type=jnp.float32)
    ilo = _bf16_round(lax.bitcast_convert_type(e_lo, jnp.int32))
    ihi = _bf16_round(lax.bitcast_convert_type(e_hi, jnp.int32))
    word = jnp.bitwise_or(jnp.bitwise_and(ilo, jnp.int32(-65536)),
                          lax.shift_right_logical(ihi, 16))
    out_ref[...] = lax.bitcast_convert_type(word, jnp.float32)


def _make_precompute(nblk, n_cols):
    last = (n_cols + BW - 1) // BW - 1
    in_specs = [
        pl.BlockSpec((NF, BW), functools.partial(
            lambda u, i: (0, jnp.minimum(nblk * u + i, last)), u))
        for u in range(8)
    ]
    in_specs.append(pl.BlockSpec((LW, LW), lambda i: (0, 0)))
    return pl.pallas_call(
        _precompute_body,
        grid=(nblk,),
        in_specs=in_specs,
        out_specs=pl.BlockSpec((BW, LW), lambda i: (i, 0)),
        out_shape=jax.ShapeDtypeStruct((nblk * BW, LW), jnp.float32),
    )


_precompute_p = _make_precompute(NBLK_P, NP_ROWS)
_precompute_q = _make_precompute(NBLK_Q, NQ_ROWS)


def _to_packed(v, qs):
    u = jnp.where(v >= qs, 1, 0)
    for k in range(2, 8):
        u = u + jnp.where(v >= k * qs, 1, 0)
    return v - u * qs


def _make_gather(qs):
    @functools.partial(
        pl.kernel,
        mesh=_mesh,
        out_type=jax.ShapeDtypeStruct((B, LW), jnp.float32),
        scratch_types=[
            pltpu.VMEM((BPW,), jnp.int32),
            pltpu.VMEM((CH, LW), jnp.float32),
            pltpu.VMEM((CH, LW), jnp.float32),
            pltpu.VMEM((CH, LW), jnp.float32),
            pltpu.VMEM((CH, LW), jnp.float32),
            pltpu.SemaphoreType.DMA,
            pltpu.SemaphoreType.DMA,
        ],
    )
    def _g(idx_hbm, t_hbm, row_out, idx_v, b0, b1, b2, b3, gsem, wsem):
        wid = lax.axis_index("s") * NC + lax.axis_index("c")
        base = wid * BPW
        pltpu.sync_copy(idx_hbm.at[0, pl.ds(base, BPW)], idx_v)
        for s in range(BPW // 16):
            sl = pl.ds(s * 16, 16)
            idx_v[sl] = _to_packed(idx_v[sl], qs)
        bufs = (b0, b1, b2, b3)
        gcps = [
            pltpu.async_copy(t_hbm.at[idx_v.at[pl.ds(c * CH, CH)]],
                             bufs[c], gsem)
            for c in range(NCH)
        ]
        for cp in gcps:
            cp.wait()
        wcps = [
            pltpu.async_copy(bufs[c], row_out.at[pl.ds(base + c * CH, CH)],
                             wsem)
            for c in range(NCH)
        ]
        for cp in wcps:
            cp.wait()

    return _g


_gather_p = _make_gather(QS_P)
_gather_q = _make_gather(QS_Q)


NB = 4
BM = B // NB


def _chunk8(v, qs):
    u = jnp.where(v >= qs, 1, 0)
    for k in range(2, 8):
        u = u + jnp.where(v >= k * qs, 1, 0)
    return u


def _unpack_select(idx_row, row_ref, qs):
    chunks = lax.broadcasted_iota(jnp.int32, (8, 1), 0)
    oh = (chunks == _chunk8(idx_row, qs)).astype(jnp.float32)
    lane_blk = lax.broadcasted_iota(jnp.int32, (8, LW), 1) // NF
    is_top = (chunks < 4)
    r_top = ((lane_blk == chunks) & is_top).astype(jnp.float32)
    r_low = ((lane_blk == chunks - 4) & ~is_top).astype(jnp.float32)
    m_top = lax.dot_general(oh, r_top,
                            dimension_numbers=(((0,), (0,)), ((), ())),
                            preferred_element_type=jnp.float32)
    m_low = lax.dot_general(oh, r_low,
                            dimension_numbers=(((0,), (0,)), ((), ())),
                            preferred_element_type=jnp.float32)
    iw = lax.bitcast_convert_type(row_ref[...], jnp.int32)
    top = lax.bitcast_convert_type(
        jnp.bitwise_and(iw, jnp.int32(-65536)), jnp.float32)
    low = lax.bitcast_convert_type(lax.shift_left(iw, 16), jnp.float32)
    return top * m_top + low * m_low


def _mlp_body(u_ref, v_ref, p_ref, q_ref, s_ref, b1_ref, w2t_ref,
              b2_ref, out_ref):
    xp = _unpack_select(u_ref[...], p_ref, QS_P)
    xq = _unpack_select(v_ref[...], q_ref, QS_Q)
    h = lax.dot_general(xp + xq, s_ref[...],
                        dimension_numbers=(((1,), (0,)), ((), ())),
                        preferred_element_type=jnp.float32)
    h = jnp.maximum(h + b1_ref[...], 0.0)
    o = lax.dot_general(w2t_ref[...], h,
                        dimension_numbers=(((1,), (1,)), ((), ())),
                        preferred_element_type=jnp.float32)
    out_ref[...] = o + b2_ref[...]


_mlp = pl.pallas_call(
    _mlp_body,
    grid=(NB,),
    in_specs=[
        pl.BlockSpec((1, BM), lambda i: (0, i)),
        pl.BlockSpec((1, BM), lambda i: (0, i)),
        pl.BlockSpec((BM, LW), lambda i: (i, 0)),
        pl.BlockSpec((BM, LW), lambda i: (i, 0)),
        pl.BlockSpec((LW, NF), lambda i: (0, 0)),
        pl.BlockSpec((1, NF), lambda i: (0, 0)),
        pl.BlockSpec((1, NF), lambda i: (0, 0)),
        pl.BlockSpec((1, 1), lambda i: (0, 0)),
    ],
    out_specs=pl.BlockSpec((1, BM), lambda i: (0, i)),
    out_shape=jax.ShapeDtypeStruct((1, B), jnp.float32),
)


def kernel(user, product, P_table, Q_table, W1, b1, W2, b2):
    pt = P_table.T
    qt = Q_table.T
    eye4 = jnp.eye(4, dtype=jnp.float32)
    w4a = jnp.kron(eye4, W1[:NF]).astype(jnp.bfloat16)
    w4b = jnp.kron(eye4, W1[NF:]).astype(jnp.bfloat16)
    uidx = user.reshape(1, B).astype(jnp.int32)
    pidx = product.reshape(1, B).astype(jnp.int32)
    b_sc = _precompute_q(qt, qt, qt, qt, qt, qt, qt, qt, w4b)
    qrow = _gather_q(pidx, b_sc)
    a_sc = _precompute_p(pt, pt, pt, pt, pt, pt, pt, pt, w4a)
    prow = _gather_p(uidx, a_sc)
    sel = jnp.tile(jnp.eye(NF, dtype=jnp.float32), (4, 1))
    out_t = _mlp(uidx, pidx, prow, qrow, sel, b1.reshape(1, NF),
                 W2.reshape(1, NF), b2.reshape(1, 1))
    return out_t.T


# speedup vs baseline: 1.0829x; 1.0404x over previous
import functools

import jax
import jax.numpy as jnp
from jax import lax
from jax.experimental import pallas as pl
from jax.experimental.pallas import tpu as pltpu
from jax.experimental.pallas import tpu_sc as plsc

B = 16384
NF = 32
LW = 128
BW = 8192

NP_ROWS = 1000000
NQ_ROWS = 100000
NBLK_P = 16
NBLK_Q = 2
QS_P = NBLK_P * BW
QS_Q = NBLK_Q * BW

NC = 2
NS = 16
NW = NC * NS
BPW = B // NW
CH = 128
NCH = BPW // CH

_mesh = plsc.VectorSubcoreMesh(core_axis_name="c", subcore_axis_name="s")


def _bf16_round(iw):
    return iw + jnp.int32(0x8000)


def _precompute_body(x0, x1, x2, x3, x4, x5, x6, x7, w_ref, out_ref):
    lo4 = jnp.concatenate(
        [x0[...], x1[...], x2[...], x3[...]], axis=0).astype(jnp.bfloat16)
    hi4 = jnp.concatenate(
        [x4[...], x5[...], x6[...], x7[...]], axis=0).astype(jnp.bfloat16)
    e_lo = lax.dot_general(
        lo4, w_ref[...], dimension_numbers=(((0,), (0,)), ((), ())),
        preferred_element_type=jnp.float32)
    e_hi = lax.dot_general(
        hi4, w_ref[...], dimension_numbers=(((0,), (0,)), ((), ())),
        preferred_element_type=jnp.float32)
    ilo = _bf16_round(lax.bitcast_convert_type(e_lo, jnp.int32))
    ihi = _bf16_round(lax.bitcast_convert_type(e_hi, jnp.int32))
    word = jnp.bitwise_or(jnp.bitwise_and(ilo, jnp.int32(-65536)),
                          lax.shift_right_logical(ihi, 16))
    out_ref[...] = lax.bitcast_convert_type(word, jnp.float32)


def _make_precompute(nblk, n_cols):
    last = (n_cols + BW - 1) // BW - 1
    in_specs = [
        pl.BlockSpec((NF, BW), functools.partial(
            lambda u, i: (0, jnp.minimum(nblk * u + i, last)), u))
        for u in range(8)
    ]
    in_specs.append(pl.BlockSpec((LW, LW), lambda i: (0, 0)))
    return pl.pallas_call(
        _precompute_body,
        grid=(nblk,),
        in_specs=in_specs,
        out_specs=pl.BlockSpec((BW, LW), lambda i: (i, 0)),
        out_shape=jax.ShapeDtypeStruct((nblk * BW, LW), jnp.float32),
    )


_precompute_p = _make_precompute(NBLK_P, NP_ROWS)
_precompute_q = _make_precompute(NBLK_Q, NQ_ROWS)


def _to_packed(v, qs):
    u = jnp.where(v >= qs, 1, 0)
    for k in range(2, 8):
        u = u + jnp.where(v >= k * qs, 1, 0)
    return v - u * qs


def _make_gather(qs):
    @functools.partial(
        pl.kernel,
        mesh=_mesh,
        out_type=jax.ShapeDtypeStruct((B, LW), jnp.float32),
        scratch_types=[
            pltpu.VMEM((BPW,), jnp.int32),
            pltpu.VMEM((CH, LW), jnp.float32),
            pltpu.VMEM((CH, LW), jnp.float32),
            pltpu.VMEM((CH, LW), jnp.float32),
            pltpu.VMEM((CH, LW), jnp.float32),
            pltpu.SemaphoreType.DMA,
            pltpu.SemaphoreType.DMA,
        ],
    )
    def _g(idx_hbm, t_hbm, row_out, idx_v, b0, b1, b2, b3, gsem, wsem):
        wid = lax.axis_index("s") * NC + lax.axis_index("c")
        base = wid * BPW
        pltpu.sync_copy(idx_hbm.at[0, pl.ds(base, BPW)], idx_v)
        for s in range(BPW // 16):
            sl = pl.ds(s * 16, 16)
            idx_v[sl] = _to_packed(idx_v[sl], qs)
        bufs = (b0, b1, b2, b3)
        gcps = [
            pltpu.async_copy(t_hbm.at[idx_v.at[pl.ds(c * CH, CH)]],
                             bufs[c], gsem)
            for c in range(NCH)
        ]
        for cp in gcps:
            cp.wait()
        wcps = [
            pltpu.async_copy(bufs[c], row_out.at[pl.ds(base + c * CH, CH)],
                             wsem)
            for c in range(NCH)
        ]
        for cp in wcps:
            cp.wait()

    return _g


_gather_p = _make_gather(QS_P)
_gather_q = _make_gather(QS_Q)


NB = 4
BM = B // NB


def _chunk8(v, qs):
    u = jnp.where(v >= qs, 1, 0)
    for k in range(2, 8):
        u = u + jnp.where(v >= k * qs, 1, 0)
    return u


def _unpack_select(idx_row, row_ref, qs):
    chunks = lax.broadcasted_iota(jnp.int32, (8, 1), 0)
    oh = (chunks == _chunk8(idx_row, qs)).astype(jnp.float32)
    lane_blk = lax.broadcasted_iota(jnp.int32, (8, LW), 1) // NF
    is_top = (chunks < 4)
    r_top = ((lane_blk == chunks) & is_top).astype(jnp.float32)
    r_low = ((lane_blk == chunks - 4) & ~is_top).astype(jnp.float32)
    m_top = lax.dot_general(oh, r_top,
                            dimension_numbers=(((0,), (0,)), ((), ())),
                            preferred_element_type=jnp.float32)
    m_low = lax.dot_general(oh, r_low,
                            dimension_numbers=(((0,), (0,)), ((), ())),
                            preferred_element_type=jnp.float32)
    iw = lax.bitcast_convert_type(row_ref[...], jnp.int32)
    top = lax.bitcast_convert_type(
        jnp.bitwise_and(iw, jnp.int32(-65536)), jnp.float32)
    low = lax.bitcast_convert_type(lax.shift_left(iw, 16), jnp.float32)
    return top * m_top + low * m_low


def _mlp_body(u_ref, v_ref, p_ref, q_ref, s_ref, b1_ref, w2t_ref,
              b2_ref, out_ref):
    xp = _unpack_select(u_ref[...], p_ref, QS_P)
    xq = _unpack_select(v_ref[...], q_ref, QS_Q)
    h = lax.dot_general(xp + xq, s_ref[...],
                        dimension_numbers=(((1,), (0,)), ((), ())),
                        preferred_element_type=jnp.float32)
    h = jnp.maximum(h + b1_ref[...], 0.0)
    o = lax.dot_general(w2t_ref[...], h,
                        dimension_numbers=(((1,), (1,)), ((), ())),
                        preferred_element_type=jnp.float32)
    out_ref[...] = o + b2_ref[...]


_mlp = pl.pallas_call(
    _mlp_body,
    grid=(NB,),
    in_specs=[
        pl.BlockSpec((1, BM), lambda i: (0, i)),
        pl.BlockSpec((1, BM), lambda i: (0, i)),
        pl.BlockSpec((BM, LW), lambda i: (i, 0)),
        pl.BlockSpec((BM, LW), lambda i: (i, 0)),
        pl.BlockSpec((LW, NF), lambda i: (0, 0)),
        pl.BlockSpec((1, NF), lambda i: (0, 0)),
        pl.BlockSpec((1, NF), lambda i: (0, 0)),
        pl.BlockSpec((1, 1), lambda i: (0, 0)),
    ],
    out_specs=pl.BlockSpec((1, BM), lambda i: (0, i)),
    out_shape=jax.ShapeDtypeStruct((1, B), jnp.float32),
)


def kernel(user, product, P_table, Q_table, W1, b1, W2, b2):
    pt = P_table.T
    qt = Q_table.T
    eye4 = jnp.eye(4, dtype=jnp.float32)
    w4a = jnp.kron(eye4, W1[:NF]).astype(jnp.bfloat16)
    w4b = jnp.kron(eye4, W1[NF:]).astype(jnp.bfloat16)
    uidx = user.reshape(1, B).astype(jnp.int32)
    pidx = product.reshape(1, B).astype(jnp.int32)
    b_sc = _precompute_q(qt, qt, qt, qt, qt, qt, qt, qt, w4b)
    qrow = _gather_q(pidx, b_sc)
    a_sc = _precompute_p(pt, pt, pt, pt, pt, pt, pt, pt, w4a)
    prow = _gather_p(uidx, a_sc)
    sel = jnp.tile(jnp.eye(NF, dtype=jnp.float32), (4, 1))
    out_t = _mlp(uidx, pidx, prow, qrow, sel, b1.reshape(1, NF),
                 W2.reshape(1, NF), b2.reshape(1, 1))
    return out_t.T
